# baseline (speedup 1.0000x reference)
import functools
import math

import jax
import jax.numpy as jnp
from jax.experimental import pallas as pl
from jax.experimental.pallas import tpu as pltpu

HIDDEN = 2048
N_HEADS = 16
N_KV_HEADS = 4
HEAD_DIM = HIDDEN // N_HEADS
GROUPS = N_HEADS // N_KV_HEADS
ROPE_THETA = 10000.0
BP = 512
BQ = 512
KC = 512
SCALE = 1.0 / math.sqrt(HEAD_DIM)


def _qkv_rope_kernel(h_ref, wq_ref, wk_ref, wv_ref, cos_ref, sin_ref,
                     q_ref, k_ref, v_ref):
    h = h_ref[...]
    cos = cos_ref[...][:, None, :]
    sin = sin_ref[...][:, None, :]

    def rope(x, n_heads):
        x = x.reshape(x.shape[0], n_heads, HEAD_DIM)
        x1 = x[..., : HEAD_DIM // 2]
        x2 = x[..., HEAD_DIM // 2:]
        out = jnp.concatenate([x1 * cos - x2 * sin, x2 * cos + x1 * sin],
                              axis=-1)
        return out.reshape(x.shape[0], n_heads * HEAD_DIM)

    q = jnp.dot(h, wq_ref[...], preferred_element_type=jnp.float32)
    k = jnp.dot(h, wk_ref[...], preferred_element_type=jnp.float32)
    v = jnp.dot(h, wv_ref[...], preferred_element_type=jnp.float32)
    q_ref[...] = rope(q, N_HEADS).astype(jnp.bfloat16)
    k_ref[...] = rope(k, N_KV_HEADS).astype(jnp.bfloat16)
    v_ref[...] = v.astype(jnp.bfloat16)


def _attn_kernel(q_ref, k_ref, v_ref, wo_ref, out_ref, *, seq):
    i = pl.program_id(0)
    h = pl.program_id(1)
    q = q_ref[...]

    row = i * BQ + jax.lax.broadcasted_iota(jnp.int32, (BQ, KC), 0)
    col_local = jax.lax.broadcasted_iota(jnp.int32, (BQ, KC), 1)

    def body(kb, carry):
        m, l, acc = carry
        k = k_ref[pl.ds(kb * KC, KC), :]
        v = v_ref[pl.ds(kb * KC, KC), :]
        s = jax.lax.dot_general(q, k, (((1,), (1,)), ((), ())),
                                preferred_element_type=jnp.float32) * SCALE
        s = jnp.where(kb * KC + col_local <= row, s, -jnp.inf)
        m_new = jnp.maximum(m, jnp.max(s, axis=-1, keepdims=True))
        alpha = jnp.exp(m - m_new)
        p = jnp.exp(s - m_new)
        l = l * alpha + jnp.sum(p, axis=-1, keepdims=True)
        acc = acc * alpha + jnp.dot(p.astype(jnp.bfloat16), v,
                                    preferred_element_type=jnp.float32)
        return m_new, l, acc

    m0 = jnp.full((BQ, 1), -jnp.inf, dtype=jnp.float32)
    l0 = jnp.zeros((BQ, 1), dtype=jnp.float32)
    a0 = jnp.zeros((BQ, HEAD_DIM), dtype=jnp.float32)
    n_chunks = (i * BQ) // KC + BQ // KC
    m, l, acc = jax.lax.fori_loop(0, n_chunks, body, (m0, l0, a0))
    attn = (acc / l).astype(jnp.bfloat16)
    contrib = jnp.dot(attn, wo_ref[...], preferred_element_type=jnp.float32)

    @pl.when(h == 0)
    def _():
        out_ref[...] = contrib

    @pl.when(h > 0)
    def _():
        out_ref[...] += contrib


@jax.jit
def kernel(hidden_states, position_ids, Wq, Wk, Wv, Wo):
    b, seq, _ = hidden_states.shape
    h2 = hidden_states.reshape(b * seq, HIDDEN).astype(jnp.bfloat16)
    Wq = Wq.astype(jnp.bfloat16)
    Wk = Wk.astype(jnp.bfloat16)
    Wv = Wv.astype(jnp.bfloat16)
    Wo = Wo.astype(jnp.bfloat16)

    pos = position_ids.reshape(b * seq).astype(jnp.float32)
    inv_freq = 1.0 / (ROPE_THETA ** (
        jnp.arange(0, HEAD_DIM, 2, dtype=jnp.float32) / HEAD_DIM))
    freqs = pos[:, None] * inv_freq[None, :]
    cos = jnp.cos(freqs)
    sin = jnp.sin(freqs)

    n_p = (b * seq) // BP
    q2, k2, v2 = pl.pallas_call(
        _qkv_rope_kernel,
        grid=(n_p,),
        in_specs=[
            pl.BlockSpec((BP, HIDDEN), lambda i: (i, 0)),
            pl.BlockSpec((HIDDEN, N_HEADS * HEAD_DIM), lambda i: (0, 0)),
            pl.BlockSpec((HIDDEN, N_KV_HEADS * HEAD_DIM), lambda i: (0, 0)),
            pl.BlockSpec((HIDDEN, N_KV_HEADS * HEAD_DIM), lambda i: (0, 0)),
            pl.BlockSpec((BP, HEAD_DIM // 2), lambda i: (i, 0)),
            pl.BlockSpec((BP, HEAD_DIM // 2), lambda i: (i, 0)),
        ],
        out_specs=[
            pl.BlockSpec((BP, N_HEADS * HEAD_DIM), lambda i: (i, 0)),
            pl.BlockSpec((BP, N_KV_HEADS * HEAD_DIM), lambda i: (i, 0)),
            pl.BlockSpec((BP, N_KV_HEADS * HEAD_DIM), lambda i: (i, 0)),
        ],
        out_shape=[
            jax.ShapeDtypeStruct((b * seq, N_HEADS * HEAD_DIM), jnp.bfloat16),
            jax.ShapeDtypeStruct((b * seq, N_KV_HEADS * HEAD_DIM), jnp.bfloat16),
            jax.ShapeDtypeStruct((b * seq, N_KV_HEADS * HEAD_DIM), jnp.bfloat16),
        ],
        compiler_params=pltpu.CompilerParams(
            dimension_semantics=("parallel",)),
    )(h2, Wq, Wk, Wv, cos, sin)

    n_q = (b * seq) // BQ
    out = pl.pallas_call(
        functools.partial(_attn_kernel, seq=b * seq),
        grid=(n_q, N_HEADS),
        in_specs=[
            pl.BlockSpec((BQ, HEAD_DIM), lambda i, h: (i, h)),
            pl.BlockSpec((b * seq, HEAD_DIM), lambda i, h: (0, h // GROUPS)),
            pl.BlockSpec((b * seq, HEAD_DIM), lambda i, h: (0, h // GROUPS)),
            pl.BlockSpec((HEAD_DIM, HIDDEN), lambda i, h: (h, 0)),
        ],
        out_specs=pl.BlockSpec((BQ, HIDDEN), lambda i, h: (i, 0)),
        out_shape=jax.ShapeDtypeStruct((b * seq, HIDDEN), jnp.float32),
        compiler_params=pltpu.CompilerParams(
            dimension_semantics=("parallel", "arbitrary")),
    )(q2, k2, v2, Wo)

    return out.reshape(b, seq, HIDDEN)


# speedup vs baseline: 1.3611x; 1.3611x over previous
import functools
import math

import jax
import jax.numpy as jnp
from jax.experimental import pallas as pl
from jax.experimental.pallas import tpu as pltpu

HIDDEN = 2048
N_HEADS = 16
N_KV_HEADS = 4
HEAD_DIM = HIDDEN // N_HEADS
GROUPS = N_HEADS // N_KV_HEADS
ROPE_THETA = 10000.0
BP = 512
BQ = 512
KC = 512
SCALE = 1.0 / math.sqrt(HEAD_DIM)


def _qkv_rope_kernel(h_ref, wq_ref, wk_ref, wv_ref, cos_ref, sin_ref,
                     q_ref, k_ref, v_ref):
    h = h_ref[...]
    cos = cos_ref[...][:, None, :]
    sin = sin_ref[...][:, None, :]

    def rope(x, n_heads):
        x = x.reshape(x.shape[0], n_heads, HEAD_DIM)
        x1 = x[..., : HEAD_DIM // 2]
        x2 = x[..., HEAD_DIM // 2:]
        out = jnp.concatenate([x1 * cos - x2 * sin, x2 * cos + x1 * sin],
                              axis=-1)
        return out.reshape(x.shape[0], n_heads * HEAD_DIM)

    q = jnp.dot(h, wq_ref[...], preferred_element_type=jnp.float32)
    k = jnp.dot(h, wk_ref[...], preferred_element_type=jnp.float32)
    v = jnp.dot(h, wv_ref[...], preferred_element_type=jnp.float32)
    q_ref[...] = rope(q, N_HEADS)
    k_ref[...] = rope(k, N_KV_HEADS)
    v_ref[...] = v


def _attn_kernel(q_ref, k_ref, v_ref, wo_ref, out_ref, acc_ref, *, seq):
    i = pl.program_id(0)
    h = pl.program_id(1)
    q = q_ref[...]

    row = i * BQ + jax.lax.broadcasted_iota(jnp.int32, (BQ, KC), 0)
    col_local = jax.lax.broadcasted_iota(jnp.int32, (BQ, KC), 1)

    def body(kb, carry):
        m, l, acc = carry
        k = k_ref[pl.ds(kb * KC, KC), :]
        v = v_ref[pl.ds(kb * KC, KC), :]
        s = jax.lax.dot_general(q, k, (((1,), (1,)), ((), ())),
                                preferred_element_type=jnp.float32) * SCALE
        s = jnp.where(kb * KC + col_local <= row, s, -jnp.inf)
        m_new = jnp.maximum(m, jnp.max(s, axis=-1, keepdims=True))
        alpha = jnp.exp(m - m_new)
        p = jnp.exp(s - m_new)
        l = l * alpha + jnp.sum(p, axis=-1, keepdims=True)
        acc = acc * alpha + jnp.dot(p, v, preferred_element_type=jnp.float32)
        return m_new, l, acc

    m0 = jnp.full((BQ, 1), -jnp.inf, dtype=jnp.float32)
    l0 = jnp.zeros((BQ, 1), dtype=jnp.float32)
    a0 = jnp.zeros((BQ, HEAD_DIM), dtype=jnp.float32)
    n_chunks = (i * BQ) // KC + BQ // KC
    m, l, acc = jax.lax.fori_loop(0, n_chunks, body, (m0, l0, a0))
    acc_ref[:, pl.ds(h * HEAD_DIM, HEAD_DIM)] = acc / l

    @pl.when(h == N_HEADS - 1)
    def _():
        out_ref[...] = jnp.dot(acc_ref[...], wo_ref[...],
                               preferred_element_type=jnp.float32)


@jax.jit
def kernel(hidden_states, position_ids, Wq, Wk, Wv, Wo):
    b, seq, _ = hidden_states.shape
    h2 = hidden_states.reshape(b * seq, HIDDEN)

    pos = position_ids.reshape(b * seq).astype(jnp.float32)
    inv_freq = 1.0 / (ROPE_THETA ** (
        jnp.arange(0, HEAD_DIM, 2, dtype=jnp.float32) / HEAD_DIM))
    freqs = pos[:, None] * inv_freq[None, :]
    cos = jnp.cos(freqs)
    sin = jnp.sin(freqs)

    n_p = (b * seq) // BP
    q2, k2, v2 = pl.pallas_call(
        _qkv_rope_kernel,
        grid=(n_p,),
        in_specs=[
            pl.BlockSpec((BP, HIDDEN), lambda i: (i, 0)),
            pl.BlockSpec((HIDDEN, N_HEADS * HEAD_DIM), lambda i: (0, 0)),
            pl.BlockSpec((HIDDEN, N_KV_HEADS * HEAD_DIM), lambda i: (0, 0)),
            pl.BlockSpec((HIDDEN, N_KV_HEADS * HEAD_DIM), lambda i: (0, 0)),
            pl.BlockSpec((BP, HEAD_DIM // 2), lambda i: (i, 0)),
            pl.BlockSpec((BP, HEAD_DIM // 2), lambda i: (i, 0)),
        ],
        out_specs=[
            pl.BlockSpec((BP, N_HEADS * HEAD_DIM), lambda i: (i, 0)),
            pl.BlockSpec((BP, N_KV_HEADS * HEAD_DIM), lambda i: (i, 0)),
            pl.BlockSpec((BP, N_KV_HEADS * HEAD_DIM), lambda i: (i, 0)),
        ],
        out_shape=[
            jax.ShapeDtypeStruct((b * seq, N_HEADS * HEAD_DIM), jnp.float32),
            jax.ShapeDtypeStruct((b * seq, N_KV_HEADS * HEAD_DIM), jnp.float32),
            jax.ShapeDtypeStruct((b * seq, N_KV_HEADS * HEAD_DIM), jnp.float32),
        ],
        compiler_params=pltpu.CompilerParams(
            dimension_semantics=("parallel",)),
    )(h2, Wq, Wk, Wv, cos, sin)

    n_q = (b * seq) // BQ
    out = pl.pallas_call(
        functools.partial(_attn_kernel, seq=b * seq),
        grid=(n_q, N_HEADS),
        in_specs=[
            pl.BlockSpec((BQ, HEAD_DIM), lambda i, h: (i, h)),
            pl.BlockSpec((b * seq, HEAD_DIM), lambda i, h: (0, h // GROUPS)),
            pl.BlockSpec((b * seq, HEAD_DIM), lambda i, h: (0, h // GROUPS)),
            pl.BlockSpec((N_HEADS * HEAD_DIM, HIDDEN), lambda i, h: (0, 0)),
        ],
        out_specs=pl.BlockSpec((BQ, HIDDEN), lambda i, h: (i, 0)),
        out_shape=jax.ShapeDtypeStruct((b * seq, HIDDEN), jnp.float32),
        scratch_shapes=[pltpu.VMEM((BQ, HIDDEN), jnp.float32)],
        compiler_params=pltpu.CompilerParams(
            dimension_semantics=("parallel", "arbitrary")),
    )(q2, k2, v2, Wo)

    return out.reshape(b, seq, HIDDEN)
